# baseline (speedup 1.0000x reference)
import functools

import jax
import jax.numpy as jnp
from jax import lax
from jax.experimental import pallas as pl
from jax.experimental.pallas import tpu as pltpu
from jax.experimental.pallas import tpu_sc as plsc

_NUM_E = 1024
_DIM = 64
_COMMITMENT_COST = 0.25


def _idx_block(x_ref, emb_ref, idx_ref, eb_ref, e2_ref):
    @pl.when(pl.program_id(0) == 0)
    def _():
        emb = emb_ref[...]
        eb_ref[...] = emb.astype(jnp.bfloat16)
        e2_ref[...] = jnp.sum(emb * emb, axis=1)[None, :]

    x = x_ref[...]
    m2 = jax.lax.dot_general(
        (-2.0 * x).astype(jnp.bfloat16), eb_ref[...], (((1,), (1,)), ((), ())),
        preferred_element_type=jnp.float32)
    x2 = jnp.sum(x * x, axis=1, keepdims=True)
    d = (x2 + e2_ref[...]) + m2

    dmin = jnp.min(d, axis=1, keepdims=True)
    col = jax.lax.broadcasted_iota(jnp.int32, (1, _NUM_E), 1)
    s = jnp.where(d == dmin, col, _NUM_E)
    idx_ref[...] = jnp.min(s, axis=1, keepdims=True)


@functools.partial(jax.jit, static_argnames=("block_rows",))
def _vq_idx(x, embeddings, block_rows=4096):
    n = x.shape[0]
    grid = (n // block_rows,)
    return pl.pallas_call(
        _idx_block,
        grid=grid,
        in_specs=[
            pl.BlockSpec((block_rows, _DIM), lambda i: (i, 0)),
            pl.BlockSpec((_NUM_E, _DIM), lambda i: (0, 0)),
        ],
        out_specs=pl.BlockSpec((block_rows, 1), lambda i: (i, 0)),
        out_shape=jax.ShapeDtypeStruct((n, 1), jnp.int32),
        scratch_shapes=[
            pltpu.VMEM((_NUM_E, _DIM), jnp.bfloat16),
            pltpu.VMEM((1, _NUM_E), jnp.float32),
        ],
    )(x, embeddings)


_info = plsc.get_sparse_core_info()
_NC, _NS, _L = _info.num_cores, _info.num_subcores, _info.num_lanes
_NW = _NC * _NS


_CHUNK = 128


def _make_sc_stage(n):
    b_per_w = n // _NW
    n_chunks = b_per_w // _CHUNK
    mesh = plsc.VectorSubcoreMesh(core_axis_name="c", subcore_axis_name="s")

    @functools.partial(
        pl.kernel, mesh=mesh,
        out_type=[
            jax.ShapeDtypeStruct((n, _DIM), jnp.float32),
            jax.ShapeDtypeStruct((_NW, _L), jnp.float32),
        ],
        scratch_types=[
            pltpu.VMEM((_CHUNK,), jnp.int32),
            pltpu.VMEM((_CHUNK, 128), jnp.float32),
            pltpu.VMEM((_CHUNK, _DIM), jnp.float32),
            pltpu.VMEM((_L,), jnp.float32),
            pltpu.SemaphoreType.DMA,
        ],
    )
    def sc_stage(table_hbm, idx_hbm, x_hbm, latent_hbm, losses_hbm,
                 idx_v, rows_v, x_v, acc_v, sem):
        wid = lax.axis_index("s") * _NC + lax.axis_index("c")
        base = wid * b_per_w

        def chunk_step(k, acc):
            cbase = base + k * _CHUNK
            pltpu.sync_copy(idx_hbm.at[pl.ds(cbase, _CHUNK)], idx_v)
            pltpu.async_copy(table_hbm.at[idx_v], rows_v, sem).wait()
            pltpu.sync_copy(x_hbm.at[pl.ds(cbase, _CHUNK), :], x_v)

            def row_step(r, a):
                for c in range(_DIM // _L):
                    xv = x_v[r, pl.ds(c * _L, _L)]
                    ev = rows_v[r, pl.ds(c * _L, _L)]
                    diff = ev - xv
                    x_v[r, pl.ds(c * _L, _L)] = xv + diff
                    a = a + diff * diff
                return a

            acc = lax.fori_loop(0, _CHUNK, row_step, acc)
            pltpu.sync_copy(x_v, latent_hbm.at[pl.ds(cbase, _CHUNK), :])
            return acc

        acc = lax.fori_loop(0, n_chunks, chunk_step,
                            jnp.zeros((_L,), jnp.float32))
        acc_v[...] = acc
        pltpu.sync_copy(acc_v, losses_hbm.at[wid])

    return sc_stage


def kernel(inputs, embeddings):
    x = inputs.reshape(-1, _DIM)
    n = x.shape[0]
    idx = _vq_idx(x, embeddings).reshape(-1)
    table = jnp.pad(embeddings.astype(jnp.bfloat16).astype(jnp.float32),
                    ((0, 0), (0, 128 - _DIM)))
    latent, losses = _make_sc_stage(n)(table, idx, x)
    mean_sq = jnp.sum(losses) / jnp.float32(n * _DIM)
    loss = _COMMITMENT_COST * mean_sq + mean_sq
    return loss, latent.reshape(inputs.shape)


# speedup vs baseline: 1.3787x; 1.3787x over previous
import functools

import jax
import jax.numpy as jnp
from jax.experimental import pallas as pl
from jax.experimental.pallas import tpu as pltpu

_NUM_E = 1024
_DIM = 64
_COMMITMENT_COST = 0.25


def _vq_block(x_ref, emb_ref, latent_ref, loss_ref, eb_ref, e2_ref):
    @pl.when(pl.program_id(0) == 0)
    def _():
        emb = emb_ref[...]
        eb_ref[...] = emb.astype(jnp.bfloat16)
        e2_ref[...] = jnp.sum(emb * emb, axis=1)[None, :]
        loss_ref[...] = jnp.zeros_like(loss_ref)

    x = x_ref[...]
    eb = eb_ref[...]
    e2 = e2_ref[...]

    m2 = jax.lax.dot_general(
        (-2.0 * x).astype(jnp.bfloat16), eb, (((1,), (1,)), ((), ())),
        preferred_element_type=jnp.float32)
    x2 = jnp.sum(x * x, axis=1, keepdims=True)
    d = (x2 + e2) + m2

    dmin = jnp.min(d, axis=1, keepdims=True)
    col = jax.lax.broadcasted_iota(jnp.int32, (1, _NUM_E), 1)
    s = jnp.where(d == dmin, col, _NUM_E)
    idx = jnp.min(s, axis=1, keepdims=True)
    onehot = (s == idx).astype(jnp.bfloat16)
    e = jax.lax.dot_general(
        onehot, eb, (((1,), (0,)), ((), ())),
        preferred_element_type=jnp.float32)

    latent_ref[...] = x + (e - x)
    loss_ref[...] += jnp.sum((e - x) ** 2, keepdims=True).reshape(1, 1)


@functools.partial(jax.jit, static_argnames=("block_rows",))
def _vq(inputs, embeddings, block_rows=4096):
    x = inputs.reshape(-1, _DIM)
    n = x.shape[0]
    grid = (n // block_rows,)
    latent, loss_sum = pl.pallas_call(
        _vq_block,
        grid=grid,
        in_specs=[
            pl.BlockSpec((block_rows, _DIM), lambda i: (i, 0)),
            pl.BlockSpec((_NUM_E, _DIM), lambda i: (0, 0)),
        ],
        out_specs=[
            pl.BlockSpec((block_rows, _DIM), lambda i: (i, 0)),
            pl.BlockSpec((1, 1), lambda i: (0, 0)),
        ],
        out_shape=[
            jax.ShapeDtypeStruct((n, _DIM), jnp.float32),
            jax.ShapeDtypeStruct((1, 1), jnp.float32),
        ],
        scratch_shapes=[
            pltpu.VMEM((_NUM_E, _DIM), jnp.bfloat16),
            pltpu.VMEM((1, _NUM_E), jnp.float32),
        ],
    )(x, embeddings)
    mean_sq = loss_sum[0, 0] / jnp.float32(n * _DIM)
    loss = _COMMITMENT_COST * mean_sq + mean_sq
    return loss, latent.reshape(inputs.shape)


def kernel(inputs, embeddings):
    return _vq(inputs, embeddings)
